# baseline (speedup 1.0000x reference)
import functools
from fractions import Fraction
from math import factorial

import numpy as np
import jax
import jax.numpy as jnp
from jax import lax
from jax.experimental import pallas as pl
from jax.experimental.pallas import tpu as pltpu
from jax.experimental.pallas import tpu_sc as plsc

N_NODES = 10000
N_EDGES = 160000
RS_IN = [(8, 0), (4, 1)]
RS_OUT = [(8, 0), (4, 1)]
DIM_IN = 20
DIM_OUT = 20
N_R = 176
H = 100
N_BASIS = 10

NC, NS = 2, 16
NW = NC * NS
CHUNK = 128
E_PAD = 163840
NSPLIT = 2
E_SPLIT = E_PAD // NSPLIT
EPW = E_SPLIT // NW
NCH = EPW // CHUNK
N_PAD = 10240
ROWS_PER_SUB = N_PAD // NS
FPAD = 32
EB = 8192


def _f(n):
    return factorial(round(n))


def _su2_cg(j1, m1, j2, m2, j3, m3):
    if m3 != m1 + m2:
        return 0.0
    vmin = int(max([-j1 + j2 + m3, -j1 + m1, 0]))
    vmax = int(min([j2 + j3 + m1, j3 - j1 + j2, j3 + m3]))
    C = ((2.0 * j3 + 1.0) * Fraction(
        _f(j3 + j1 - j2) * _f(j3 - j1 + j2) * _f(j1 + j2 - j3) * _f(j3 + m3) * _f(j3 - m3),
        _f(j1 + j2 + j3 + 1) * _f(j1 - m1) * _f(j1 + m1) * _f(j2 - m2) * _f(j2 + m2))) ** 0.5
    S = 0
    for v in range(vmin, vmax + 1):
        S += (-1) ** int(v + j2 + m2) * Fraction(
            _f(j2 + j3 + m1 - v) * _f(j1 - m1 + v),
            _f(v) * _f(j3 - j1 + j2 - v) * _f(j3 + m3 - v) * _f(v + j1 - j2 - m3))
    return float(C * S)


def _su2_cg_mat(j1, j2, j3):
    mat = np.zeros((2 * j1 + 1, 2 * j2 + 1, 2 * j3 + 1))
    if abs(j1 - j2) <= j3 <= j1 + j2:
        for m1 in range(-j1, j1 + 1):
            for m2 in range(-j2, j2 + 1):
                if abs(m1 + m2) <= j3:
                    mat[j1 + m1, j2 + m2, j3 + m1 + m2] = _su2_cg(j1, m1, j2, m2, j3, m1 + m2)
    return mat


def _real_to_complex(l):
    q = np.zeros((2 * l + 1, 2 * l + 1), dtype=np.complex128)
    for m in range(-l, 0):
        q[l + m, l + abs(m)] = 1 / np.sqrt(2)
        q[l + m, l - abs(m)] = -1j / np.sqrt(2)
    q[l, l] = 1.0
    for m in range(1, l + 1):
        q[l + m, l + abs(m)] = (-1) ** m / np.sqrt(2)
        q[l + m, l - abs(m)] = 1j * (-1) ** m / np.sqrt(2)
    return (-1j) ** l * q


def _so3_cg(l1, l2, l3):
    Q1 = _real_to_complex(l1)
    Q2 = _real_to_complex(l2)
    Q3 = _real_to_complex(l3)
    C = _su2_cg_mat(l1, l2, l3).astype(np.complex128)
    C = np.einsum('ij,kl,mn,ikn->jlm', Q1, Q2, np.conj(Q3.T), C)
    return np.real(C).astype(np.float32)


def _norm_coefs():
    nc = np.zeros((len(RS_OUT), len(RS_IN)), dtype=np.float32)
    for i, (mo, lo) in enumerate(RS_OUT):
        ns = sum(mi * (2 * min(lo, li) + 1) for mi, li in RS_IN)
        for j in range(len(RS_IN)):
            nc[i, j] = np.sqrt(4 * np.pi) * np.sqrt(2 * lo + 1) / np.sqrt(ns)
    return nc


def _build_msg_constants():
    norm = _norm_coefs()
    y_off = {0: 0, 1: 1, 2: 4}
    ny = 9
    npd = ny * DIM_IN
    slot_r, slot_out, c2_cols = [], [], []
    r_off = 0
    for i, (mo, lo) in enumerate(RS_OUT):
        f_off = 0
        for j, (mi, li) in enumerate(RS_IN):
            di = mi * (2 * li + 1)
            lfs = list(range(abs(lo - li), lo + li + 1))
            nk = len(lfs)
            for k, lf in enumerate(lfs):
                C = _so3_cg(li, lf, lo)
                for u in range(mo):
                    for v in range(mi):
                        for o in range(2 * lo + 1):
                            slot_r.append(r_off + u * (mi * nk) + v * nk + k)
                            slot_out.append(u if i == 0 else 8 + u * 3 + o)
                            col = np.zeros((npd,), np.float64)
                            for f in range(2 * lf + 1):
                                for ii in range(2 * li + 1):
                                    p = (y_off[lf] + f) * DIM_IN + f_off + v * (2 * li + 1) + ii
                                    col[p] += C[ii, f, o] * norm[i, j]
                            c2_cols.append(col)
            f_off += di
            r_off += mo * mi * nk
    S = len(slot_r)
    Q1 = np.zeros((N_R, S), np.float32)
    Q3 = np.zeros((S, FPAD), np.float32)
    for s in range(S):
        Q1[slot_r[s], s] = 1.0
        Q3[s, slot_out[s]] = 1.0
    C2 = np.stack(c2_cols, axis=1).astype(np.float32)
    A = np.zeros((FPAD, npd), np.float32)
    B = np.zeros((ny, npd), np.float32)
    for p in range(npd):
        A[p % DIM_IN, p] = 1.0
        B[p // DIM_IN, p] = 1.0
    mono = [(0, 0), (0, 1), (0, 2), (0, 3), (1, 2), (2, 3), (3, 3), (1, 3), (1, 1), (2, 2)]
    nm = len(mono)
    M1 = np.zeros((4, nm), np.float32)
    M2 = np.zeros((4, nm), np.float32)
    for q, (a, b) in enumerate(mono):
        M1[a, q] = 1.0
        M2[b, q] = 1.0
    c1 = 0.4886025119029199
    c2a = 1.0925484305920792
    YC = np.zeros((nm, ny), np.float32)
    YC[0, 0] = 0.28209479177387814
    YC[mono.index((0, 2)), 1] = c1
    YC[mono.index((0, 3)), 2] = c1
    YC[mono.index((0, 1)), 3] = c1
    YC[mono.index((1, 2)), 4] = c2a
    YC[mono.index((2, 3)), 5] = c2a
    YC[mono.index((3, 3)), 6] = 3.0 * 0.31539156525252005
    YC[0, 6] = -0.31539156525252005
    YC[mono.index((1, 3)), 7] = c2a
    YC[mono.index((1, 1)), 8] = 0.5462742152960396
    YC[mono.index((2, 2)), 8] = -0.5462742152960396
    YB = YC @ B
    return A, YB, M1, M2, Q1, C2, Q3


_A, _YB, _M1, _M2, _Q1, _C2, _Q3 = _build_msg_constants()
NSLOT = _Q1.shape[1]


def _tc_msg_body(d_ref, rel_ref, fj_ref, w0, b0, w1, b1, w2, b2, w3, b3,
                 a_ref, yb_ref, m1_ref, m2_ref, c2_ref, q3_ref, out_ref):
    f32 = jnp.float32
    dot = functools.partial(jax.lax.dot_general,
                            dimension_numbers=(((1,), (0,)), ((), ())),
                            preferred_element_type=f32)
    d = d_ref[...]
    step = (3.2 - 0.7) / (N_BASIS - 1)
    centers = 0.7 + step * lax.broadcasted_iota(jnp.int32, (1, N_BASIS), 1).astype(f32)
    t = (d - centers) * (1.0 / step)
    basis = jnp.exp(-(t * t))
    h = dot(basis, w0[...]) + b0[...]
    h = h * (1.0 / (1.0 + jnp.exp(-h)))
    h = dot(h, w1[...]) + b1[...]
    h = h * (1.0 / (1.0 + jnp.exp(-h)))
    h = dot(h, w2[...]) + b2[...]
    h = h * (1.0 / (1.0 + jnp.exp(-h)))
    Rx = dot(h, w3[...]) + b3[...]

    rel = rel_ref[...]
    rr = rel * rel
    r = jnp.sqrt(rr[:, 0:1] + rr[:, 1:2] + rr[:, 2:3])
    inv = 1.0 / jnp.maximum(r, 1e-9)
    z = jnp.concatenate([jnp.full(d.shape, 1.0, f32), rel * inv], axis=1)
    m = dot(z, m1_ref[...]) * dot(z, m2_ref[...])
    gy = dot(m, yb_ref[...])

    fj = fj_ref[...]
    G = dot(fj, a_ref[...]) * gy
    out_ref[...] = dot(Rx * dot(G, c2_ref[...]), q3_ref[...])


def _tc_msg(d2, rel, fj, w0, b0, w1, b1, w2, b2, w3, b3, consts,
            interpret=False):
    full = lambda s: pl.BlockSpec(s, lambda i: (0, 0))
    in_specs = [
        pl.BlockSpec((EB, 1), lambda i: (i, 0)),
        pl.BlockSpec((EB, 3), lambda i: (i, 0)),
        pl.BlockSpec((EB, FPAD), lambda i: (i, 0)),
        full((N_BASIS, H)), full((1, H)),
        full((H, H)), full((1, H)),
        full((H, H)), full((1, H)),
        full((H, NSLOT)), full((1, NSLOT)),
    ] + [full(c.shape) for c in consts]
    return pl.pallas_call(
        _tc_msg_body,
        grid=(d2.shape[0] // EB,),
        in_specs=in_specs,
        out_specs=pl.BlockSpec((EB, FPAD), lambda i: (i, 0)),
        out_shape=jax.ShapeDtypeStruct((d2.shape[0], FPAD), jnp.float32),
        interpret=interpret,
    )(d2, rel, fj, w0, b0.reshape(1, H), w1, b1.reshape(1, H),
      w2, b2.reshape(1, H), w3, b3.reshape(1, NSLOT), *consts)


def _tc_combine_body(p_ref, out_ref):
    acc = p_ref[0, :, :DIM_OUT]
    for q in range(1, NSPLIT * NC):
        acc = acc + p_ref[q, :, :DIM_OUT]
    out_ref[...] = acc


def _tc_combine(partials, interpret=False):
    rb = 1000
    return pl.pallas_call(
        _tc_combine_body,
        grid=(N_NODES // rb,),
        in_specs=[pl.BlockSpec((NSPLIT * NC, rb, FPAD), lambda i: (0, i, 0))],
        out_specs=pl.BlockSpec((rb, DIM_OUT), lambda i: (i, 0)),
        out_shape=jax.ShapeDtypeStruct((N_NODES, DIM_OUT), jnp.float32),
        interpret=interpret,
    )(partials)


@functools.lru_cache(maxsize=1)
def _sc_kernels():
    mesh = plsc.VectorSubcoreMesh(core_axis_name="c", subcore_axis_name="s")

    sc_params = pltpu.CompilerParams(use_tc_tiling_on_sc=False)

    @functools.partial(
        pl.kernel, mesh=mesh, compiler_params=sc_params,
        out_type=jax.ShapeDtypeStruct((E_SPLIT, FPAD), jnp.float32),
        scratch_types=[
            pltpu.VMEM((NCH, CHUNK), jnp.int32),
            pltpu.VMEM((CHUNK, FPAD), jnp.float32),
            pltpu.VMEM((CHUNK, FPAD), jnp.float32),
            pltpu.SemaphoreType.DMA,
            pltpu.SemaphoreType.DMA,
            pltpu.SemaphoreType.DMA,
            pltpu.SemaphoreType.DMA,
        ],
    )
    def _sc_gather(x_hbm, src_hbm, out_hbm, idx_v, rows_v0, rows_v1, g0, g1, w0, w1):
        c = lax.axis_index("c")
        s = lax.axis_index("s")
        wid = s * NC + c
        base = wid * EPW
        pltpu.sync_copy(src_hbm.at[wid], idx_v)
        rows = (rows_v0, rows_v1)
        gsem = (g0, g1)
        wsem = (w0, w1)
        gd = [None, None]
        wd = [None, None]
        for j in range(NCH + 1):
            b = j & 1
            if j < NCH:
                if wd[b] is not None:
                    wd[b].wait()
                gd[b] = pltpu.async_copy(x_hbm.at[idx_v.at[j]], rows[b], gsem[b])
            if j >= 1:
                pb = (j - 1) & 1
                gd[pb].wait()
                wd[pb] = pltpu.async_copy(
                    rows[pb], out_hbm.at[pl.ds(base + (j - 1) * CHUNK, CHUNK)], wsem[pb])
        for b in (0, 1):
            if wd[b] is not None:
                wd[b].wait()

    @functools.partial(
        pl.kernel, mesh=mesh, compiler_params=sc_params,
        out_type=jax.ShapeDtypeStruct((NC, N_PAD, FPAD), jnp.float32),
        scratch_types=[
            pltpu.VMEM((NCH, CHUNK), jnp.int32),
            pltpu.VMEM((CHUNK, FPAD), jnp.float32),
            pltpu.VMEM((CHUNK, FPAD), jnp.float32),
            pltpu.VMEM_SHARED((N_PAD, FPAD), jnp.float32),
            pltpu.SemaphoreType.DMA,
            pltpu.SemaphoreType.DMA,
            pltpu.SemaphoreType.DMA,
            pltpu.SemaphoreType.DMA,
        ],
    )
    def _sc_scatter(msg_hbm, dst_hbm, zeros_hbm, out_hbm, idx_v, rows_v0, rows_v1,
                    acc_sh, l0, l1, a0, a1):
        c = lax.axis_index("c")
        s = lax.axis_index("s")
        wid = s * NC + c
        base = wid * EPW
        row0 = s * ROWS_PER_SUB
        zd = pltpu.async_copy(zeros_hbm.at[pl.ds(row0, ROWS_PER_SUB)],
                              acc_sh.at[pl.ds(row0, ROWS_PER_SUB)], a0)
        pltpu.sync_copy(dst_hbm.at[wid], idx_v)
        zd.wait()
        plsc.subcore_barrier()
        rows = (rows_v0, rows_v1)
        lsem = (l0, l1)
        asem = (a0, a1)
        ld = [None, None]
        ad = [None, None]
        ld[0] = pltpu.async_copy(msg_hbm.at[pl.ds(base, CHUNK)], rows[0], lsem[0])
        for j in range(NCH):
            b = j & 1
            nb = 1 - b
            if j + 1 < NCH:
                if ad[nb] is not None:
                    ad[nb].wait()
                ld[nb] = pltpu.async_copy(
                    msg_hbm.at[pl.ds(base + (j + 1) * CHUNK, CHUNK)], rows[nb], lsem[nb])
            ld[b].wait()
            ad[b] = pltpu.async_copy(rows[b], acc_sh.at[idx_v.at[j]], asem[b], add=True)
        for b in (0, 1):
            if ad[b] is not None:
                ad[b].wait()
        plsc.subcore_barrier()
        pltpu.sync_copy(acc_sh.at[pl.ds(row0, ROWS_PER_SUB)],
                        out_hbm.at[c, pl.ds(row0, ROWS_PER_SUB)])

    return _sc_gather, _sc_scatter


def kernel(x, edge_index, abs_distances, rel_vec, W0, b0, W1, b1, W2, b2, W3, b3):
    f32 = jnp.float32
    pad = E_PAD - N_EDGES
    src = edge_index[0].astype(jnp.int32)
    dst = edge_index[1].astype(jnp.int32)
    src3 = jnp.concatenate([src, jnp.zeros((pad,), jnp.int32)]).reshape(NSPLIT * NW, NCH, CHUNK)
    dump = N_NODES + (jnp.arange(pad, dtype=jnp.int32) % (N_PAD - N_NODES))
    dst3 = jnp.concatenate([dst, dump]).reshape(NSPLIT * NW, NCH, CHUNK)
    d2 = jnp.concatenate([abs_distances, jnp.ones((pad,), f32)]).reshape(E_PAD, 1)
    rel = jnp.concatenate([rel_vec, jnp.ones((pad, 3), f32)], axis=0)
    x_pad = x if FPAD == DIM_IN else jnp.pad(x, ((0, 0), (0, FPAD - DIM_IN)))
    zeros = jnp.zeros((N_PAD, FPAD), f32)
    consts = tuple(jnp.asarray(m) for m in (_A, _YB, _M1, _M2, _C2, _Q3))
    q1 = jnp.asarray(_Q1)
    w3x = W3 @ q1
    b3x = b3 @ q1

    sc_gather, sc_scatter = _sc_kernels()
    parts = []
    for t in range(NSPLIT):
        e0 = t * E_SPLIT
        fj = sc_gather(x_pad, src3[t * NW:(t + 1) * NW])
        msg = _tc_msg(d2[e0:e0 + E_SPLIT], rel[e0:e0 + E_SPLIT], fj,
                      W0, b0, W1, b1, W2, b2, w3x, b3x, consts)
        parts.append(sc_scatter(msg, dst3[t * NW:(t + 1) * NW], zeros))
    return _tc_combine(jnp.concatenate(parts, axis=0))


# speedup vs baseline: 1.0080x; 1.0080x over previous
import functools
from fractions import Fraction
from math import factorial

import numpy as np
import jax
import jax.numpy as jnp
from jax import lax
from jax.experimental import pallas as pl
from jax.experimental.pallas import tpu as pltpu
from jax.experimental.pallas import tpu_sc as plsc

N_NODES = 10000
N_EDGES = 160000
RS_IN = [(8, 0), (4, 1)]
RS_OUT = [(8, 0), (4, 1)]
DIM_IN = 20
DIM_OUT = 20
N_R = 176
H = 100
N_BASIS = 10

NC, NS = 2, 16
NW = NC * NS
CHUNK = 128
E_PAD = 163840
NSPLIT = 2
SPLITS = (98304, 65536)
NCHS = tuple(s // (NW * CHUNK) for s in SPLITS)
N_PAD = 10240
ROWS_PER_SUB = N_PAD // NS
FPAD = 32
EB = 4096


def _f(n):
    return factorial(round(n))


def _su2_cg(j1, m1, j2, m2, j3, m3):
    if m3 != m1 + m2:
        return 0.0
    vmin = int(max([-j1 + j2 + m3, -j1 + m1, 0]))
    vmax = int(min([j2 + j3 + m1, j3 - j1 + j2, j3 + m3]))
    C = ((2.0 * j3 + 1.0) * Fraction(
        _f(j3 + j1 - j2) * _f(j3 - j1 + j2) * _f(j1 + j2 - j3) * _f(j3 + m3) * _f(j3 - m3),
        _f(j1 + j2 + j3 + 1) * _f(j1 - m1) * _f(j1 + m1) * _f(j2 - m2) * _f(j2 + m2))) ** 0.5
    S = 0
    for v in range(vmin, vmax + 1):
        S += (-1) ** int(v + j2 + m2) * Fraction(
            _f(j2 + j3 + m1 - v) * _f(j1 - m1 + v),
            _f(v) * _f(j3 - j1 + j2 - v) * _f(j3 + m3 - v) * _f(v + j1 - j2 - m3))
    return float(C * S)


def _su2_cg_mat(j1, j2, j3):
    mat = np.zeros((2 * j1 + 1, 2 * j2 + 1, 2 * j3 + 1))
    if abs(j1 - j2) <= j3 <= j1 + j2:
        for m1 in range(-j1, j1 + 1):
            for m2 in range(-j2, j2 + 1):
                if abs(m1 + m2) <= j3:
                    mat[j1 + m1, j2 + m2, j3 + m1 + m2] = _su2_cg(j1, m1, j2, m2, j3, m1 + m2)
    return mat


def _real_to_complex(l):
    q = np.zeros((2 * l + 1, 2 * l + 1), dtype=np.complex128)
    for m in range(-l, 0):
        q[l + m, l + abs(m)] = 1 / np.sqrt(2)
        q[l + m, l - abs(m)] = -1j / np.sqrt(2)
    q[l, l] = 1.0
    for m in range(1, l + 1):
        q[l + m, l + abs(m)] = (-1) ** m / np.sqrt(2)
        q[l + m, l - abs(m)] = 1j * (-1) ** m / np.sqrt(2)
    return (-1j) ** l * q


def _so3_cg(l1, l2, l3):
    Q1 = _real_to_complex(l1)
    Q2 = _real_to_complex(l2)
    Q3 = _real_to_complex(l3)
    C = _su2_cg_mat(l1, l2, l3).astype(np.complex128)
    C = np.einsum('ij,kl,mn,ikn->jlm', Q1, Q2, np.conj(Q3.T), C)
    return np.real(C).astype(np.float32)


def _norm_coefs():
    nc = np.zeros((len(RS_OUT), len(RS_IN)), dtype=np.float32)
    for i, (mo, lo) in enumerate(RS_OUT):
        ns = sum(mi * (2 * min(lo, li) + 1) for mi, li in RS_IN)
        for j in range(len(RS_IN)):
            nc[i, j] = np.sqrt(4 * np.pi) * np.sqrt(2 * lo + 1) / np.sqrt(ns)
    return nc


def _build_msg_constants():
    norm = _norm_coefs()
    y_off = {0: 0, 1: 1, 2: 4}
    ny = 9
    npd = ny * DIM_IN
    slot_r, slot_out, c2_cols = [], [], []
    r_off = 0
    for i, (mo, lo) in enumerate(RS_OUT):
        f_off = 0
        for j, (mi, li) in enumerate(RS_IN):
            di = mi * (2 * li + 1)
            lfs = list(range(abs(lo - li), lo + li + 1))
            nk = len(lfs)
            for k, lf in enumerate(lfs):
                C = _so3_cg(li, lf, lo)
                for u in range(mo):
                    for v in range(mi):
                        for o in range(2 * lo + 1):
                            slot_r.append(r_off + u * (mi * nk) + v * nk + k)
                            slot_out.append(u if i == 0 else 8 + u * 3 + o)
                            col = np.zeros((npd,), np.float64)
                            for f in range(2 * lf + 1):
                                for ii in range(2 * li + 1):
                                    p = (y_off[lf] + f) * DIM_IN + f_off + v * (2 * li + 1) + ii
                                    col[p] += C[ii, f, o] * norm[i, j]
                            c2_cols.append(col)
            f_off += di
            r_off += mo * mi * nk
    S = len(slot_r)
    Q1 = np.zeros((N_R, S), np.float32)
    Q3 = np.zeros((S, FPAD), np.float32)
    for s in range(S):
        Q1[slot_r[s], s] = 1.0
        Q3[s, slot_out[s]] = 1.0
    C2 = np.stack(c2_cols, axis=1).astype(np.float32)
    A = np.zeros((FPAD, npd), np.float32)
    B = np.zeros((ny, npd), np.float32)
    for p in range(npd):
        A[p % DIM_IN, p] = 1.0
        B[p // DIM_IN, p] = 1.0
    mono = [(0, 0), (0, 1), (0, 2), (0, 3), (1, 2), (2, 3), (3, 3), (1, 3), (1, 1), (2, 2)]
    nm = len(mono)
    M1 = np.zeros((4, nm), np.float32)
    M2 = np.zeros((4, nm), np.float32)
    for q, (a, b) in enumerate(mono):
        M1[a, q] = 1.0
        M2[b, q] = 1.0
    c1 = 0.4886025119029199
    c2a = 1.0925484305920792
    YC = np.zeros((nm, ny), np.float32)
    YC[0, 0] = 0.28209479177387814
    YC[mono.index((0, 2)), 1] = c1
    YC[mono.index((0, 3)), 2] = c1
    YC[mono.index((0, 1)), 3] = c1
    YC[mono.index((1, 2)), 4] = c2a
    YC[mono.index((2, 3)), 5] = c2a
    YC[mono.index((3, 3)), 6] = 3.0 * 0.31539156525252005
    YC[0, 6] = -0.31539156525252005
    YC[mono.index((1, 3)), 7] = c2a
    YC[mono.index((1, 1)), 8] = 0.5462742152960396
    YC[mono.index((2, 2)), 8] = -0.5462742152960396
    YB = YC @ B
    return A, YB, M1, M2, Q1, C2, Q3


_A, _YB, _M1, _M2, _Q1, _C2, _Q3 = _build_msg_constants()
NSLOT = _Q1.shape[1]


def _tc_msg_body(d_ref, rel_ref, fj_ref, w0, b0, w1, b1, w2, b2, w3, b3,
                 a_ref, yb_ref, m1_ref, m2_ref, c2_ref, q3_ref, out_ref):
    f32 = jnp.float32
    dot = functools.partial(jax.lax.dot_general,
                            dimension_numbers=(((1,), (0,)), ((), ())),
                            preferred_element_type=f32)
    d = d_ref[...]
    step = (3.2 - 0.7) / (N_BASIS - 1)
    centers = 0.7 + step * lax.broadcasted_iota(jnp.int32, (1, N_BASIS), 1).astype(f32)
    t = (d - centers) * (1.0 / step)
    basis = jnp.exp(-(t * t))
    h = dot(basis, w0[...]) + b0[...]
    h = h * (1.0 / (1.0 + jnp.exp(-h)))
    h = dot(h, w1[...]) + b1[...]
    h = h * (1.0 / (1.0 + jnp.exp(-h)))
    h = dot(h, w2[...]) + b2[...]
    h = h * (1.0 / (1.0 + jnp.exp(-h)))
    Rx = dot(h, w3[...]) + b3[...]

    rel = rel_ref[...]
    rr = rel * rel
    r = jnp.sqrt(rr[:, 0:1] + rr[:, 1:2] + rr[:, 2:3])
    inv = 1.0 / jnp.maximum(r, 1e-9)
    z = jnp.concatenate([jnp.full(d.shape, 1.0, f32), rel * inv], axis=1)
    m = dot(z, m1_ref[...]) * dot(z, m2_ref[...])
    gy = dot(m, yb_ref[...])

    fj = fj_ref[...]
    G = dot(fj, a_ref[...]) * gy
    out_ref[...] = dot(Rx * dot(G, c2_ref[...]), q3_ref[...])


def _tc_msg(d2, rel, fj, w0, b0, w1, b1, w2, b2, w3, b3, consts,
            interpret=False):
    full = lambda s: pl.BlockSpec(s, lambda i: (0, 0))
    in_specs = [
        pl.BlockSpec((EB, 1), lambda i: (i, 0)),
        pl.BlockSpec((EB, 3), lambda i: (i, 0)),
        pl.BlockSpec((EB, FPAD), lambda i: (i, 0)),
        full((N_BASIS, H)), full((1, H)),
        full((H, H)), full((1, H)),
        full((H, H)), full((1, H)),
        full((H, NSLOT)), full((1, NSLOT)),
    ] + [full(c.shape) for c in consts]
    return pl.pallas_call(
        _tc_msg_body,
        grid=(d2.shape[0] // EB,),
        in_specs=in_specs,
        out_specs=pl.BlockSpec((EB, FPAD), lambda i: (i, 0)),
        out_shape=jax.ShapeDtypeStruct((d2.shape[0], FPAD), jnp.float32),
        interpret=interpret,
    )(d2, rel, fj, w0, b0.reshape(1, H), w1, b1.reshape(1, H),
      w2, b2.reshape(1, H), w3, b3.reshape(1, NSLOT), *consts)


def _tc_combine_body(p_ref, out_ref):
    acc = p_ref[0, :, :DIM_OUT]
    for q in range(1, NSPLIT * NC):
        acc = acc + p_ref[q, :, :DIM_OUT]
    out_ref[...] = acc


def _tc_combine(partials, interpret=False):
    rb = 1000
    return pl.pallas_call(
        _tc_combine_body,
        grid=(N_NODES // rb,),
        in_specs=[pl.BlockSpec((NSPLIT * NC, rb, FPAD), lambda i: (0, i, 0))],
        out_specs=pl.BlockSpec((rb, DIM_OUT), lambda i: (i, 0)),
        out_shape=jax.ShapeDtypeStruct((N_NODES, DIM_OUT), jnp.float32),
        interpret=interpret,
    )(partials)


@functools.lru_cache(maxsize=None)
def _sc_kernels(NCH):
    EPW = NCH * CHUNK
    E_SPLIT = NW * EPW
    mesh = plsc.VectorSubcoreMesh(core_axis_name="c", subcore_axis_name="s")

    sc_params = pltpu.CompilerParams(use_tc_tiling_on_sc=False)

    @functools.partial(
        pl.kernel, mesh=mesh, compiler_params=sc_params,
        out_type=jax.ShapeDtypeStruct((E_SPLIT, FPAD), jnp.float32),
        scratch_types=[
            pltpu.VMEM((NCH, CHUNK), jnp.int32),
            pltpu.VMEM((CHUNK, FPAD), jnp.float32),
            pltpu.VMEM((CHUNK, FPAD), jnp.float32),
            pltpu.SemaphoreType.DMA,
            pltpu.SemaphoreType.DMA,
            pltpu.SemaphoreType.DMA,
            pltpu.SemaphoreType.DMA,
        ],
    )
    def _sc_gather(x_hbm, src_hbm, out_hbm, idx_v, rows_v0, rows_v1, g0, g1, w0, w1):
        c = lax.axis_index("c")
        s = lax.axis_index("s")
        wid = s * NC + c
        base = wid * EPW
        pltpu.sync_copy(src_hbm.at[wid], idx_v)
        rows = (rows_v0, rows_v1)
        gsem = (g0, g1)
        wsem = (w0, w1)
        gd = [None, None]
        wd = [None, None]
        for j in range(NCH + 1):
            b = j & 1
            if j < NCH:
                if wd[b] is not None:
                    wd[b].wait()
                gd[b] = pltpu.async_copy(x_hbm.at[idx_v.at[j]], rows[b], gsem[b])
            if j >= 1:
                pb = (j - 1) & 1
                gd[pb].wait()
                wd[pb] = pltpu.async_copy(
                    rows[pb], out_hbm.at[pl.ds(base + (j - 1) * CHUNK, CHUNK)], wsem[pb])
        for b in (0, 1):
            if wd[b] is not None:
                wd[b].wait()

    @functools.partial(
        pl.kernel, mesh=mesh, compiler_params=sc_params,
        out_type=jax.ShapeDtypeStruct((NC, N_PAD, FPAD), jnp.float32),
        scratch_types=[
            pltpu.VMEM((NCH, CHUNK), jnp.int32),
            pltpu.VMEM((CHUNK, FPAD), jnp.float32),
            pltpu.VMEM((CHUNK, FPAD), jnp.float32),
            pltpu.VMEM_SHARED((N_PAD, FPAD), jnp.float32),
            pltpu.SemaphoreType.DMA,
            pltpu.SemaphoreType.DMA,
            pltpu.SemaphoreType.DMA,
            pltpu.SemaphoreType.DMA,
        ],
    )
    def _sc_scatter(msg_hbm, dst_hbm, zeros_hbm, out_hbm, idx_v, rows_v0, rows_v1,
                    acc_sh, l0, l1, a0, a1):
        c = lax.axis_index("c")
        s = lax.axis_index("s")
        wid = s * NC + c
        base = wid * EPW
        row0 = s * ROWS_PER_SUB
        zd = pltpu.async_copy(zeros_hbm.at[pl.ds(row0, ROWS_PER_SUB)],
                              acc_sh.at[pl.ds(row0, ROWS_PER_SUB)], a0)
        pltpu.sync_copy(dst_hbm.at[wid], idx_v)
        zd.wait()
        plsc.subcore_barrier()
        rows = (rows_v0, rows_v1)
        lsem = (l0, l1)
        asem = (a0, a1)
        ld = [None, None]
        ad = [None, None]
        ld[0] = pltpu.async_copy(msg_hbm.at[pl.ds(base, CHUNK)], rows[0], lsem[0])
        for j in range(NCH):
            b = j & 1
            nb = 1 - b
            if j + 1 < NCH:
                if ad[nb] is not None:
                    ad[nb].wait()
                ld[nb] = pltpu.async_copy(
                    msg_hbm.at[pl.ds(base + (j + 1) * CHUNK, CHUNK)], rows[nb], lsem[nb])
            ld[b].wait()
            ad[b] = pltpu.async_copy(rows[b], acc_sh.at[idx_v.at[j]], asem[b], add=True)
        for b in (0, 1):
            if ad[b] is not None:
                ad[b].wait()
        plsc.subcore_barrier()
        pltpu.sync_copy(acc_sh.at[pl.ds(row0, ROWS_PER_SUB)],
                        out_hbm.at[c, pl.ds(row0, ROWS_PER_SUB)])

    return _sc_gather, _sc_scatter


def kernel(x, edge_index, abs_distances, rel_vec, W0, b0, W1, b1, W2, b2, W3, b3):
    f32 = jnp.float32
    pad = E_PAD - N_EDGES
    src = edge_index[0].astype(jnp.int32)
    dst = edge_index[1].astype(jnp.int32)
    srcf = jnp.concatenate([src, jnp.zeros((pad,), jnp.int32)])
    dump = N_NODES + (jnp.arange(pad, dtype=jnp.int32) % (N_PAD - N_NODES))
    dstf = jnp.concatenate([dst, dump])
    d2 = jnp.concatenate([abs_distances, jnp.ones((pad,), f32)]).reshape(E_PAD, 1)
    rel = jnp.concatenate([rel_vec, jnp.ones((pad, 3), f32)], axis=0)
    x_pad = x if FPAD == DIM_IN else jnp.pad(x, ((0, 0), (0, FPAD - DIM_IN)))
    zeros = jnp.zeros((N_PAD, FPAD), f32)
    consts = tuple(jnp.asarray(m) for m in (_A, _YB, _M1, _M2, _C2, _Q3))
    q1 = jnp.asarray(_Q1)
    w3x = W3 @ q1
    b3x = b3 @ q1

    parts = []
    e0 = 0
    for t in range(NSPLIT):
        sz = SPLITS[t]
        nch = NCHS[t]
        sc_gather, sc_scatter = _sc_kernels(nch)
        src3 = srcf[e0:e0 + sz].reshape(NW, nch, CHUNK)
        dst3 = dstf[e0:e0 + sz].reshape(NW, nch, CHUNK)
        fj = sc_gather(x_pad, src3)
        msg = _tc_msg(d2[e0:e0 + sz], rel[e0:e0 + sz], fj,
                      W0, b0, W1, b1, W2, b2, w3x, b3x, consts)
        parts.append(sc_scatter(msg, dst3, zeros))
        e0 += sz
    return _tc_combine(jnp.concatenate(parts, axis=0))


# speedup vs baseline: 1.0295x; 1.0214x over previous
import functools
from fractions import Fraction
from math import factorial

import numpy as np
import jax
import jax.numpy as jnp
from jax import lax
from jax.experimental import pallas as pl
from jax.experimental.pallas import tpu as pltpu
from jax.experimental.pallas import tpu_sc as plsc

N_NODES = 10000
N_EDGES = 160000
RS_IN = [(8, 0), (4, 1)]
RS_OUT = [(8, 0), (4, 1)]
DIM_IN = 20
DIM_OUT = 20
N_R = 176
H = 100
N_BASIS = 10

NC, NS = 2, 16
NW = NC * NS
CHUNK = 128
E_PAD = 163840
NSPLIT = 2
SPLITS = (81920, 81920)
NCHS = tuple(s // (NW * CHUNK) for s in SPLITS)
N_PAD = 10240
ROWS_PER_SUB = N_PAD // NS
FPAD = 32
EB = 4096


def _f(n):
    return factorial(round(n))


def _su2_cg(j1, m1, j2, m2, j3, m3):
    if m3 != m1 + m2:
        return 0.0
    vmin = int(max([-j1 + j2 + m3, -j1 + m1, 0]))
    vmax = int(min([j2 + j3 + m1, j3 - j1 + j2, j3 + m3]))
    C = ((2.0 * j3 + 1.0) * Fraction(
        _f(j3 + j1 - j2) * _f(j3 - j1 + j2) * _f(j1 + j2 - j3) * _f(j3 + m3) * _f(j3 - m3),
        _f(j1 + j2 + j3 + 1) * _f(j1 - m1) * _f(j1 + m1) * _f(j2 - m2) * _f(j2 + m2))) ** 0.5
    S = 0
    for v in range(vmin, vmax + 1):
        S += (-1) ** int(v + j2 + m2) * Fraction(
            _f(j2 + j3 + m1 - v) * _f(j1 - m1 + v),
            _f(v) * _f(j3 - j1 + j2 - v) * _f(j3 + m3 - v) * _f(v + j1 - j2 - m3))
    return float(C * S)


def _su2_cg_mat(j1, j2, j3):
    mat = np.zeros((2 * j1 + 1, 2 * j2 + 1, 2 * j3 + 1))
    if abs(j1 - j2) <= j3 <= j1 + j2:
        for m1 in range(-j1, j1 + 1):
            for m2 in range(-j2, j2 + 1):
                if abs(m1 + m2) <= j3:
                    mat[j1 + m1, j2 + m2, j3 + m1 + m2] = _su2_cg(j1, m1, j2, m2, j3, m1 + m2)
    return mat


def _real_to_complex(l):
    q = np.zeros((2 * l + 1, 2 * l + 1), dtype=np.complex128)
    for m in range(-l, 0):
        q[l + m, l + abs(m)] = 1 / np.sqrt(2)
        q[l + m, l - abs(m)] = -1j / np.sqrt(2)
    q[l, l] = 1.0
    for m in range(1, l + 1):
        q[l + m, l + abs(m)] = (-1) ** m / np.sqrt(2)
        q[l + m, l - abs(m)] = 1j * (-1) ** m / np.sqrt(2)
    return (-1j) ** l * q


def _so3_cg(l1, l2, l3):
    Q1 = _real_to_complex(l1)
    Q2 = _real_to_complex(l2)
    Q3 = _real_to_complex(l3)
    C = _su2_cg_mat(l1, l2, l3).astype(np.complex128)
    C = np.einsum('ij,kl,mn,ikn->jlm', Q1, Q2, np.conj(Q3.T), C)
    return np.real(C).astype(np.float32)


def _norm_coefs():
    nc = np.zeros((len(RS_OUT), len(RS_IN)), dtype=np.float32)
    for i, (mo, lo) in enumerate(RS_OUT):
        ns = sum(mi * (2 * min(lo, li) + 1) for mi, li in RS_IN)
        for j in range(len(RS_IN)):
            nc[i, j] = np.sqrt(4 * np.pi) * np.sqrt(2 * lo + 1) / np.sqrt(ns)
    return nc


def _build_msg_constants():
    norm = _norm_coefs()
    y_off = {0: 0, 1: 1, 2: 4}
    ny = 9
    npd = ny * DIM_IN
    slot_r, slot_out, c2_cols = [], [], []
    r_off = 0
    for i, (mo, lo) in enumerate(RS_OUT):
        f_off = 0
        for j, (mi, li) in enumerate(RS_IN):
            di = mi * (2 * li + 1)
            lfs = list(range(abs(lo - li), lo + li + 1))
            nk = len(lfs)
            for k, lf in enumerate(lfs):
                C = _so3_cg(li, lf, lo)
                for u in range(mo):
                    for v in range(mi):
                        for o in range(2 * lo + 1):
                            slot_r.append(r_off + u * (mi * nk) + v * nk + k)
                            slot_out.append(u if i == 0 else 8 + u * 3 + o)
                            col = np.zeros((npd,), np.float64)
                            for f in range(2 * lf + 1):
                                for ii in range(2 * li + 1):
                                    p = (y_off[lf] + f) * DIM_IN + f_off + v * (2 * li + 1) + ii
                                    col[p] += C[ii, f, o] * norm[i, j]
                            c2_cols.append(col)
            f_off += di
            r_off += mo * mi * nk
    S = len(slot_r)
    Q1 = np.zeros((N_R, S), np.float32)
    Q3 = np.zeros((S, FPAD), np.float32)
    for s in range(S):
        Q1[slot_r[s], s] = 1.0
        Q3[s, slot_out[s]] = 1.0
    C2 = np.stack(c2_cols, axis=1).astype(np.float32)
    A = np.zeros((FPAD, npd), np.float32)
    B = np.zeros((ny, npd), np.float32)
    for p in range(npd):
        A[p % DIM_IN, p] = 1.0
        B[p // DIM_IN, p] = 1.0
    mono = [(0, 0), (0, 1), (0, 2), (0, 3), (1, 2), (2, 3), (3, 3), (1, 3), (1, 1), (2, 2)]
    nm = len(mono)
    M1 = np.zeros((4, nm), np.float32)
    M2 = np.zeros((4, nm), np.float32)
    for q, (a, b) in enumerate(mono):
        M1[a, q] = 1.0
        M2[b, q] = 1.0
    c1 = 0.4886025119029199
    c2a = 1.0925484305920792
    YC = np.zeros((nm, ny), np.float32)
    YC[0, 0] = 0.28209479177387814
    YC[mono.index((0, 2)), 1] = c1
    YC[mono.index((0, 3)), 2] = c1
    YC[mono.index((0, 1)), 3] = c1
    YC[mono.index((1, 2)), 4] = c2a
    YC[mono.index((2, 3)), 5] = c2a
    YC[mono.index((3, 3)), 6] = 3.0 * 0.31539156525252005
    YC[0, 6] = -0.31539156525252005
    YC[mono.index((1, 3)), 7] = c2a
    YC[mono.index((1, 1)), 8] = 0.5462742152960396
    YC[mono.index((2, 2)), 8] = -0.5462742152960396
    YB = YC @ B
    return A, YB, M1, M2, Q1, C2, Q3


_A, _YB, _M1, _M2, _Q1, _C2, _Q3 = _build_msg_constants()
NSLOT = _Q1.shape[1]


def _tc_msg_body(d_ref, rel_ref, fj_ref, w0, b0, w1, b1, w2, b2, w3, b3,
                 a_ref, yb_ref, m1_ref, m2_ref, c2_ref, q3_ref, out_ref):
    f32 = jnp.float32
    dot = functools.partial(jax.lax.dot_general,
                            dimension_numbers=(((1,), (0,)), ((), ())),
                            preferred_element_type=f32)
    d = d_ref[...]
    step = (3.2 - 0.7) / (N_BASIS - 1)
    centers = 0.7 + step * lax.broadcasted_iota(jnp.int32, (1, N_BASIS), 1).astype(f32)
    t = (d - centers) * (1.0 / step)
    basis = jnp.exp(-(t * t))
    h = dot(basis, w0[...]) + b0[...]
    h = h * (1.0 / (1.0 + jnp.exp(-h)))
    h = dot(h, w1[...]) + b1[...]
    h = h * (1.0 / (1.0 + jnp.exp(-h)))
    h = dot(h, w2[...]) + b2[...]
    h = h * (1.0 / (1.0 + jnp.exp(-h)))
    Rx = dot(h, w3[...]) + b3[...]

    rel = rel_ref[...]
    rr = rel * rel
    r = jnp.sqrt(rr[:, 0:1] + rr[:, 1:2] + rr[:, 2:3])
    inv = 1.0 / jnp.maximum(r, 1e-9)
    z = jnp.concatenate([jnp.full(d.shape, 1.0, f32), rel * inv], axis=1)
    m = dot(z, m1_ref[...]) * dot(z, m2_ref[...])
    gy = dot(m, yb_ref[...])

    fj = fj_ref[...]
    G = dot(fj, a_ref[...]) * gy
    out_ref[...] = dot(Rx * dot(G, c2_ref[...]), q3_ref[...])


def _tc_msg(d2, rel, fj, w0, b0, w1, b1, w2, b2, w3, b3, consts,
            blk0=0, n_edges=None, interpret=False):
    if n_edges is None:
        n_edges = fj.shape[0]
    full = lambda s: pl.BlockSpec(s, lambda i: (0, 0))
    in_specs = [
        pl.BlockSpec((EB, 1), lambda i: (blk0 + i, 0)),
        pl.BlockSpec((EB, 3), lambda i: (blk0 + i, 0)),
        pl.BlockSpec((EB, FPAD), lambda i: (i, 0)),
        full((N_BASIS, H)), full((1, H)),
        full((H, H)), full((1, H)),
        full((H, H)), full((1, H)),
        full((H, NSLOT)), full((1, NSLOT)),
    ] + [full(c.shape) for c in consts]
    return pl.pallas_call(
        _tc_msg_body,
        grid=(n_edges // EB,),
        in_specs=in_specs,
        out_specs=pl.BlockSpec((EB, FPAD), lambda i: (i, 0)),
        out_shape=jax.ShapeDtypeStruct((n_edges, FPAD), jnp.float32),
        interpret=interpret,
    )(d2, rel, fj, w0, b0.reshape(1, H), w1, b1.reshape(1, H),
      w2, b2.reshape(1, H), w3, b3.reshape(1, NSLOT), *consts)


def _tc_combine_body(*refs):
    p_refs, out_ref = refs[:-1], refs[-1]
    acc = None
    for p in p_refs:
        for q in range(NC):
            term = p[q, :, :DIM_OUT]
            acc = term if acc is None else acc + term
    out_ref[...] = acc


def _tc_combine(parts, interpret=False):
    rb = 1000
    return pl.pallas_call(
        _tc_combine_body,
        grid=(N_NODES // rb,),
        in_specs=[pl.BlockSpec((NC, rb, FPAD), lambda i: (0, i, 0))
                  for _ in parts],
        out_specs=pl.BlockSpec((rb, DIM_OUT), lambda i: (i, 0)),
        out_shape=jax.ShapeDtypeStruct((N_NODES, DIM_OUT), jnp.float32),
        interpret=interpret,
    )(*parts)


@functools.lru_cache(maxsize=None)
def _sc_kernels(NCH, ROW0):
    EPW = NCH * CHUNK
    E_SPLIT = NW * EPW
    mesh = plsc.VectorSubcoreMesh(core_axis_name="c", subcore_axis_name="s")

    sc_params = pltpu.CompilerParams(use_tc_tiling_on_sc=False)

    @functools.partial(
        pl.kernel, mesh=mesh, compiler_params=sc_params,
        out_type=jax.ShapeDtypeStruct((E_SPLIT, FPAD), jnp.float32),
        scratch_types=[
            pltpu.VMEM((NCH, CHUNK), jnp.int32),
            pltpu.VMEM((CHUNK, FPAD), jnp.float32),
            pltpu.VMEM((CHUNK, FPAD), jnp.float32),
            pltpu.SemaphoreType.DMA,
            pltpu.SemaphoreType.DMA,
            pltpu.SemaphoreType.DMA,
            pltpu.SemaphoreType.DMA,
        ],
    )
    def _sc_gather(x_hbm, src_hbm, out_hbm, idx_v, rows_v0, rows_v1, g0, g1, w0, w1):
        c = lax.axis_index("c")
        s = lax.axis_index("s")
        wid = s * NC + c
        base = wid * EPW
        pltpu.sync_copy(src_hbm.at[ROW0 + wid], idx_v)
        rows = (rows_v0, rows_v1)
        gsem = (g0, g1)
        wsem = (w0, w1)
        gd = [None, None]
        wd = [None, None]
        for j in range(NCH + 1):
            b = j & 1
            if j < NCH:
                if wd[b] is not None:
                    wd[b].wait()
                gd[b] = pltpu.async_copy(x_hbm.at[idx_v.at[j]], rows[b], gsem[b])
            if j >= 1:
                pb = (j - 1) & 1
                gd[pb].wait()
                wd[pb] = pltpu.async_copy(
                    rows[pb], out_hbm.at[pl.ds(base + (j - 1) * CHUNK, CHUNK)], wsem[pb])
        for b in (0, 1):
            if wd[b] is not None:
                wd[b].wait()

    @functools.partial(
        pl.kernel, mesh=mesh, compiler_params=sc_params,
        out_type=jax.ShapeDtypeStruct((NC, N_PAD, FPAD), jnp.float32),
        scratch_types=[
            pltpu.VMEM((NCH, CHUNK), jnp.int32),
            pltpu.VMEM((CHUNK, FPAD), jnp.float32),
            pltpu.VMEM((CHUNK, FPAD), jnp.float32),
            pltpu.VMEM_SHARED((N_PAD, FPAD), jnp.float32),
            pltpu.SemaphoreType.DMA,
            pltpu.SemaphoreType.DMA,
            pltpu.SemaphoreType.DMA,
            pltpu.SemaphoreType.DMA,
        ],
    )
    def _sc_scatter(msg_hbm, dst_hbm, zeros_hbm, out_hbm, idx_v, rows_v0, rows_v1,
                    acc_sh, l0, l1, a0, a1):
        c = lax.axis_index("c")
        s = lax.axis_index("s")
        wid = s * NC + c
        base = wid * EPW
        row0 = s * ROWS_PER_SUB
        zd = pltpu.async_copy(zeros_hbm.at[pl.ds(row0, ROWS_PER_SUB)],
                              acc_sh.at[pl.ds(row0, ROWS_PER_SUB)], a0)
        pltpu.sync_copy(dst_hbm.at[ROW0 + wid], idx_v)
        zd.wait()
        plsc.subcore_barrier()
        rows = (rows_v0, rows_v1)
        lsem = (l0, l1)
        asem = (a0, a1)
        ld = [None, None]
        ad = [None, None]
        ld[0] = pltpu.async_copy(msg_hbm.at[pl.ds(base, CHUNK)], rows[0], lsem[0])
        for j in range(NCH):
            b = j & 1
            nb = 1 - b
            if j + 1 < NCH:
                if ad[nb] is not None:
                    ad[nb].wait()
                ld[nb] = pltpu.async_copy(
                    msg_hbm.at[pl.ds(base + (j + 1) * CHUNK, CHUNK)], rows[nb], lsem[nb])
            ld[b].wait()
            ad[b] = pltpu.async_copy(rows[b], acc_sh.at[idx_v.at[j]], asem[b], add=True)
        for b in (0, 1):
            if ad[b] is not None:
                ad[b].wait()
        plsc.subcore_barrier()
        pltpu.sync_copy(acc_sh.at[pl.ds(row0, ROWS_PER_SUB)],
                        out_hbm.at[c, pl.ds(row0, ROWS_PER_SUB)])

    return _sc_gather, _sc_scatter


def kernel(x, edge_index, abs_distances, rel_vec, W0, b0, W1, b1, W2, b2, W3, b3):
    f32 = jnp.float32
    pad = E_PAD - N_EDGES
    src = edge_index[0].astype(jnp.int32)
    dst = edge_index[1].astype(jnp.int32)
    srcf = jnp.concatenate([src, jnp.zeros((pad,), jnp.int32)])
    dump = N_NODES + (jnp.arange(pad, dtype=jnp.int32) % (N_PAD - N_NODES))
    dstf = jnp.concatenate([dst, dump])
    d2 = jnp.concatenate([abs_distances, jnp.ones((pad,), f32)]).reshape(E_PAD, 1)
    rel = jnp.concatenate([rel_vec, jnp.ones((pad, 3), f32)], axis=0)
    x_pad = x if FPAD == DIM_IN else jnp.pad(x, ((0, 0), (0, FPAD - DIM_IN)))
    zeros = jnp.zeros((N_PAD, FPAD), f32)
    consts = tuple(jnp.asarray(m) for m in (_A, _YB, _M1, _M2, _C2, _Q3))
    q1 = jnp.asarray(_Q1)
    w3x = W3 @ q1
    b3x = b3 @ q1

    src3 = srcf.reshape(-1, NCHS[0], CHUNK)
    dst3 = dstf.reshape(-1, NCHS[0], CHUNK)
    parts = []
    e0 = 0
    for t in range(NSPLIT):
        sz = SPLITS[t]
        nch = NCHS[t]
        sc_gather, sc_scatter = _sc_kernels(nch, t * NW)
        fj = sc_gather(x_pad, src3)
        msg = _tc_msg(d2, rel, fj, W0, b0, W1, b1, W2, b2, w3x, b3x, consts,
                      blk0=e0 // EB, n_edges=sz)
        parts.append(sc_scatter(msg, dst3, zeros))
        e0 += sz
    return _tc_combine(parts)
